# baseline (speedup 1.0000x reference)
import jax
import jax.numpy as jnp
from jax import lax
from jax.experimental import pallas as pl
from jax.experimental.pallas import tpu as pltpu
from jax.experimental.pallas import tpu_sc as plsc

N = 10000
E = 160000
EMB = 256
H = 128
NC = 2
NS = 16
K = 80
EPT = E // NS
CH = EPT // K
NPAD = 10240
RPT = NPAD // NS
BN = 400
GRID = N // BN

_MESH = plsc.VectorSubcoreMesh(core_axis_name="c", subcore_axis_name="s")
_f32 = jnp.float32


def _zero_acc(z128_h, rows, acc, s):
    pltpu.sync_copy(z128_h, rows)
    for t in range(RPT // K):
        pltpu.sync_copy(rows, acc.at[pl.ds(s * RPT + t * K, K)])


def _dump_acc(acc, rows, out, c, s):
    for t in range(RPT // K):
        pltpu.sync_copy(acc.at[pl.ds(s * RPT + t * K, K)], rows)
        pltpu.sync_copy(rows, out.at[c, pl.ds(s * RPT + t * K, K)])


def _rpass_body(hr_h, dst_h, z128_h, sr_out, acc, rows0, rows1, dstb,
                sem0, sem1, ssem0, ssem1):
    c = lax.axis_index("c")
    s = lax.axis_index("s")
    _zero_acc(z128_h, rows0, acc, s)
    pltpu.sync_copy(dst_h.at[s], dstb)
    plsc.subcore_barrier()

    def ld(j, buf, sem):
        src = hr_h.at[pl.ds(s * EPT + j * K, K), pl.ds(c * H, H)]
        pltpu.async_copy(src, buf, sem)

    def ldw(j, buf, sem):
        src = hr_h.at[pl.ds(s * EPT + j * K, K), pl.ds(c * H, H)]
        pltpu.make_async_copy(src, buf, sem).wait()

    def si(j, buf, sem):
        pltpu.async_copy(buf, acc.at[dstb.at[j]], sem, add=True)

    def sw(j, buf, sem):
        pltpu.make_async_copy(buf, acc.at[dstb.at[j]], sem).wait()

    ld(0, rows0, sem0)
    ld(1, rows1, sem1)

    def step2(t, carry):
        j0 = 2 * t
        ldw(j0, rows0, sem0)
        si(j0, rows0, ssem0)
        ldw(j0 + 1, rows1, sem1)
        si(j0 + 1, rows1, ssem1)
        sw(j0, rows0, ssem0)
        ld(j0 + 2, rows0, sem0)
        sw(j0 + 1, rows1, ssem1)
        ld(j0 + 3, rows1, sem1)
        return carry

    lax.fori_loop(0, (CH - 3) // 2, step2, 0)
    ldw(CH - 3, rows0, sem0)
    si(CH - 3, rows0, ssem0)
    ldw(CH - 2, rows1, sem1)
    si(CH - 2, rows1, ssem1)
    sw(CH - 3, rows0, ssem0)
    ld(CH - 1, rows0, sem0)
    ldw(CH - 1, rows0, sem0)
    si(CH - 1, rows0, ssem0)
    sw(CH - 2, rows1, ssem1)
    sw(CH - 1, rows0, ssem0)
    plsc.subcore_barrier()
    _dump_acc(acc, rows0, sr_out, c, s)


_rpass = pl.kernel(
    _rpass_body,
    out_type=jax.ShapeDtypeStruct((NC, NPAD, H), _f32),
    mesh=_MESH,
    scratch_types=[
        pltpu.VMEM_SHARED((NPAD, H), _f32),
        pltpu.VMEM((K, H), _f32),
        pltpu.VMEM((K, H), _f32),
        pltpu.VMEM((CH, K), jnp.int32),
        pltpu.SemaphoreType.DMA,
        pltpu.SemaphoreType.DMA,
        pltpu.SemaphoreType.DMA,
        pltpu.SemaphoreType.DMA,
    ],
)


KC = 125
CHC = EPT // KC


def _cpass_body(dst_h, ones_h, z128_h, cnt_out, acc, rows, ones_v, dstb, sem):
    c = lax.axis_index("c")
    s = lax.axis_index("s")
    _zero_acc(z128_h, rows, acc, s)
    pltpu.sync_copy(ones_h, ones_v)
    pltpu.sync_copy(dst_h.at[s], dstb)
    plsc.subcore_barrier()

    def step(g, carry):
        for i in range(8):
            pltpu.async_copy(ones_v, acc.at[dstb.at[g * 8 + i]], sem, add=True)
        for i in range(8):
            pltpu.make_async_copy(ones_v, acc.at[dstb.at[g * 8 + i]], sem).wait()
        return carry

    lax.fori_loop(0, CHC // 8, step, 0)
    plsc.subcore_barrier()
    _dump_acc(acc, rows, cnt_out, c, s)


_cpass = pl.kernel(
    _cpass_body,
    out_type=jax.ShapeDtypeStruct((NC, NPAD, H), _f32),
    mesh=_MESH,
    scratch_types=[
        pltpu.VMEM_SHARED((NPAD, H), _f32),
        pltpu.VMEM((K, H), _f32),
        pltpu.VMEM((KC, H), _f32),
        pltpu.VMEM((CHC, KC), jnp.int32),
        pltpu.SemaphoreType.DMA,
    ],
)


def _gpass_body(tab_h, src_h, dst_h, z128_h, se_out, acc, rows0, rows1,
                idxb, dstb, sem0, sem1, ssem0, ssem1):
    c = lax.axis_index("c")
    s = lax.axis_index("s")
    _zero_acc(z128_h, rows0, acc, s)
    pltpu.sync_copy(src_h.at[pl.ds((c * NS + s) * EPT, EPT)], idxb)
    pltpu.sync_copy(dst_h.at[s], dstb)
    plsc.subcore_barrier()

    def g(j, buf, sem):
        pltpu.async_copy(tab_h.at[idxb.at[pl.ds(j * K, K)]], buf, sem)

    def gw(j, buf, sem):
        pltpu.make_async_copy(tab_h.at[idxb.at[pl.ds(j * K, K)]], buf, sem).wait()

    def si(j, buf, sem):
        pltpu.async_copy(buf, acc.at[dstb.at[j]], sem, add=True)

    def sw(j, buf, sem):
        pltpu.make_async_copy(buf, acc.at[dstb.at[j]], sem).wait()

    g(0, rows0, sem0)
    g(1, rows1, sem1)

    def step2(t, carry):
        j0 = 2 * t
        gw(j0, rows0, sem0)
        si(j0, rows0, ssem0)
        gw(j0 + 1, rows1, sem1)
        si(j0 + 1, rows1, ssem1)
        sw(j0, rows0, ssem0)
        g(j0 + 2, rows0, sem0)
        sw(j0 + 1, rows1, ssem1)
        g(j0 + 3, rows1, sem1)
        return carry

    lax.fori_loop(0, (CH - 3) // 2, step2, 0)
    gw(CH - 3, rows0, sem0)
    si(CH - 3, rows0, ssem0)
    gw(CH - 2, rows1, sem1)
    si(CH - 2, rows1, ssem1)
    sw(CH - 3, rows0, ssem0)
    g(CH - 1, rows0, sem0)
    gw(CH - 1, rows0, sem0)
    si(CH - 1, rows0, ssem0)
    sw(CH - 2, rows1, ssem1)
    sw(CH - 1, rows0, ssem0)
    plsc.subcore_barrier()
    _dump_acc(acc, rows0, se_out, c, s)


_gpass = pl.kernel(
    _gpass_body,
    out_type=jax.ShapeDtypeStruct((NC, NPAD, H), _f32),
    mesh=_MESH,
    scratch_types=[
        pltpu.VMEM_SHARED((NPAD, H), _f32),
        pltpu.VMEM((K, H), _f32),
        pltpu.VMEM((K, H), _f32),
        pltpu.VMEM((EPT,), jnp.int32),
        pltpu.VMEM((CH, K), jnp.int32),
        pltpu.SemaphoreType.DMA,
        pltpu.SemaphoreType.DMA,
        pltpu.SemaphoreType.DMA,
        pltpu.SemaphoreType.DMA,
    ],
)


def _mlp1_body(h_ref, se_ref, sr_ref, cnt_ref, wa_ref, wb_ref, wc_ref,
               b_ref, o_ref):
    inv = 1.0 / jnp.maximum(cnt_ref[:, 0:1], 1.0)
    acc = jnp.dot(h_ref[...], wa_ref[...], preferred_element_type=_f32)
    acc += jnp.dot(se_ref[0] * inv, wb_ref[0], preferred_element_type=_f32)
    acc += jnp.dot(se_ref[1] * inv, wb_ref[1], preferred_element_type=_f32)
    acc += jnp.dot(sr_ref[0] * inv, wc_ref[0], preferred_element_type=_f32)
    acc += jnp.dot(sr_ref[1] * inv, wc_ref[1], preferred_element_type=_f32)
    r = jnp.maximum(acc + b_ref[...], 0.0)
    o_ref[0] = r[:, :H]
    o_ref[1] = r[:, H:]


def _mlp2_body(h_ref, se_ref, sr_ref, cnt_ref, wa_ref, wb_ref, wc_ref,
               b_ref, o_ref):
    inv = 1.0 / jnp.maximum(cnt_ref[:, 0:1], 1.0)
    acc = jnp.dot(h_ref[0], wa_ref[0], preferred_element_type=_f32)
    acc += jnp.dot(h_ref[1], wa_ref[1], preferred_element_type=_f32)
    acc += jnp.dot(se_ref[0] * inv, wb_ref[0], preferred_element_type=_f32)
    acc += jnp.dot(se_ref[1] * inv, wb_ref[1], preferred_element_type=_f32)
    acc += jnp.dot(sr_ref[0] * inv, wc_ref[0], preferred_element_type=_f32)
    acc += jnp.dot(sr_ref[1] * inv, wc_ref[1], preferred_element_type=_f32)
    o_ref[...] = jnp.maximum(acc + b_ref[...], 0.0)


_split_spec = pl.BlockSpec((NC, BN, H), lambda i: (0, i, 0))
_wsplit_spec = pl.BlockSpec((NC, H, EMB), lambda i: (0, 0, 0))
_cnt_spec = pl.BlockSpec((BN, H), lambda i: (i, 0))

_mlp1 = pl.pallas_call(
    _mlp1_body,
    grid=(GRID,),
    in_specs=[
        pl.BlockSpec((BN, EMB), lambda i: (i, 0)),
        _split_spec,
        _split_spec,
        _cnt_spec,
        pl.BlockSpec((EMB, EMB), lambda i: (0, 0)),
        _wsplit_spec,
        _wsplit_spec,
        pl.BlockSpec((1, EMB), lambda i: (0, 0)),
    ],
    out_specs=_split_spec,
    out_shape=jax.ShapeDtypeStruct((NC, N, H), _f32),
)

_mlp2 = pl.pallas_call(
    _mlp2_body,
    grid=(GRID,),
    in_specs=[
        _split_spec,
        _split_spec,
        _split_spec,
        _cnt_spec,
        _wsplit_spec,
        _wsplit_spec,
        _wsplit_spec,
        pl.BlockSpec((1, EMB), lambda i: (0, 0)),
    ],
    out_specs=pl.BlockSpec((BN, EMB), lambda i: (i, 0)),
    out_shape=jax.ShapeDtypeStruct((N, EMB), _f32),
)


def kernel(h_e, h_r, edge_index, W1, b1, W2, b2):
    src = edge_index[0].astype(jnp.int32)
    dst = edge_index[1].astype(jnp.int32)
    src2 = jnp.concatenate([src, src + N])
    dst3 = dst.reshape(NS, CH, K)
    dst4 = dst.reshape(NS, CHC, KC)

    z128 = jnp.zeros((K, H), _f32)
    ones128 = jnp.ones((KC, H), _f32)

    tab1 = h_e.reshape(N, NC, H).transpose(1, 0, 2).reshape(NC * N, H)

    s_r = _rpass(h_r, dst3, z128)
    cnt = _cpass(dst4, ones128, z128)
    s_e1 = _gpass(tab1, src2, dst3, z128)

    w1a = W1[:EMB]
    w1b = W1[EMB:2 * EMB].reshape(NC, H, EMB)
    w1c = W1[2 * EMB:].reshape(NC, H, EMB)
    h1s = _mlp1(h_e, s_e1, s_r, cnt[0], w1a, w1b, w1c, b1.reshape(1, EMB))

    s_e2 = _gpass(h1s.reshape(NC * N, H), src2, dst3, z128)

    w2a = W2[:EMB].reshape(NC, H, EMB)
    w2b = W2[EMB:2 * EMB].reshape(NC, H, EMB)
    w2c = W2[2 * EMB:].reshape(NC, H, EMB)
    return _mlp2(h1s, s_e2, s_r, cnt[0], w2a, w2b, w2c, b2.reshape(1, EMB))


# speedup vs baseline: 1.2679x; 1.2679x over previous
import jax
import jax.numpy as jnp
from jax import lax
from jax.experimental import pallas as pl
from jax.experimental.pallas import tpu as pltpu
from jax.experimental.pallas import tpu_sc as plsc

N = 10000
E = 160000
EMB = 256
H = 128
NC = 2
NS = 16
NPAD = 10240
RPT = NPAD // NS
BN = 400
GRID = N // BN

K = 80
EPT = E // NS
CH = EPT // K
KC = 125
EPW = E // (NC * NS)
CHC = EPW // KC

_MESH = plsc.VectorSubcoreMesh(core_axis_name="c", subcore_axis_name="s")
_f32 = jnp.float32


def _zero_acc(z_h, buf, acc, s):
    pltpu.sync_copy(z_h, buf)
    for t in range(RPT // K):
        pltpu.sync_copy(buf, acc.at[pl.ds(s * RPT + t * K, K)])


def _dump_acc(acc, buf, out, c, s):
    for t in range(RPT // K):
        pltpu.sync_copy(acc.at[pl.ds(s * RPT + t * K, K)], buf)
        pltpu.sync_copy(buf, out.at[c, pl.ds(s * RPT + t * K, K)])


def _rpass_body(hr_h, dst_h, z_h, sr_out, acc, rows0, rows1, dstb,
                sem0, sem1):
    c = lax.axis_index("c")
    s = lax.axis_index("s")
    _zero_acc(z_h, rows0, acc, s)
    pltpu.sync_copy(dst_h.at[s], dstb)
    plsc.subcore_barrier()

    def ld(j, buf, sem):
        src = hr_h.at[pl.ds(s * EPT + j * K, K), pl.ds(c * H, H)]
        pltpu.async_copy(src, buf, sem)

    def ldw(j, buf, sem):
        src = hr_h.at[pl.ds(s * EPT + j * K, K), pl.ds(c * H, H)]
        pltpu.make_async_copy(src, buf, sem).wait()

    ld(0, rows0, sem0)

    def step2(t, carry):
        j0 = 2 * t
        ld(j0 + 1, rows1, sem1)
        ldw(j0, rows0, sem0)
        pltpu.sync_copy(rows0, acc.at[dstb.at[j0]], add=True)
        ld(j0 + 2, rows0, sem0)
        ldw(j0 + 1, rows1, sem1)
        pltpu.sync_copy(rows1, acc.at[dstb.at[j0 + 1]], add=True)
        return carry

    lax.fori_loop(0, (CH - 1) // 2, step2, 0)
    ldw(CH - 1, rows0, sem0)
    pltpu.sync_copy(rows0, acc.at[dstb.at[CH - 1]], add=True)
    plsc.subcore_barrier()
    _dump_acc(acc, rows0, sr_out, c, s)


_rpass = pl.kernel(
    _rpass_body,
    out_type=jax.ShapeDtypeStruct((NC, NPAD, H), _f32),
    mesh=_MESH,
    scratch_types=[
        pltpu.VMEM_SHARED((NPAD, H), _f32),
        pltpu.VMEM((K, H), _f32),
        pltpu.VMEM((K, H), _f32),
        pltpu.VMEM((CH, K), jnp.int32),
        pltpu.SemaphoreType.DMA,
        pltpu.SemaphoreType.DMA,
    ],
)


def _cpass_body(dst_h, ones_h, z_h, cnt_out, acc, rows, ones_v, dstb, sem):
    c = lax.axis_index("c")
    s = lax.axis_index("s")
    w = c * NS + s
    _zero_acc(z_h, rows, acc, s)
    pltpu.sync_copy(ones_h, ones_v)
    pltpu.sync_copy(dst_h.at[w], dstb)
    plsc.subcore_barrier()

    def step(g, carry):
        for i in range(8):
            pltpu.async_copy(ones_v, acc.at[dstb.at[g * 8 + i]], sem, add=True)
        for i in range(8):
            pltpu.make_async_copy(ones_v, acc.at[dstb.at[g * 8 + i]], sem).wait()
        return carry

    lax.fori_loop(0, CHC // 8, step, 0)
    plsc.subcore_barrier()
    _dump_acc(acc, rows, cnt_out, c, s)


_cpass = pl.kernel(
    _cpass_body,
    out_type=jax.ShapeDtypeStruct((NC, NPAD, H), _f32),
    mesh=_MESH,
    scratch_types=[
        pltpu.VMEM_SHARED((NPAD, H), _f32),
        pltpu.VMEM((K, H), _f32),
        pltpu.VMEM((KC, H), _f32),
        pltpu.VMEM((CHC, KC), jnp.int32),
        pltpu.SemaphoreType.DMA,
    ],
)


def _gpass_body(tab_h, src_h, dst_h, z_h, se_out, acc, rows0, rows1,
                idxb, dstb, sem0, sem1):
    c = lax.axis_index("c")
    s = lax.axis_index("s")
    _zero_acc(z_h, rows0, acc, s)
    pltpu.sync_copy(src_h.at[pl.ds((c * NS + s) * EPT, EPT)], idxb)
    pltpu.sync_copy(dst_h.at[s], dstb)
    plsc.subcore_barrier()

    def g(j, buf, sem):
        pltpu.async_copy(tab_h.at[idxb.at[pl.ds(j * K, K)]], buf, sem)

    def gw(j, buf, sem):
        pltpu.make_async_copy(tab_h.at[idxb.at[pl.ds(j * K, K)]], buf, sem).wait()

    g(0, rows0, sem0)

    def step2(t, carry):
        j0 = 2 * t
        g(j0 + 1, rows1, sem1)
        gw(j0, rows0, sem0)
        pltpu.sync_copy(rows0, acc.at[dstb.at[j0]], add=True)
        g(j0 + 2, rows0, sem0)
        gw(j0 + 1, rows1, sem1)
        pltpu.sync_copy(rows1, acc.at[dstb.at[j0 + 1]], add=True)
        return carry

    lax.fori_loop(0, (CH - 1) // 2, step2, 0)
    gw(CH - 1, rows0, sem0)
    pltpu.sync_copy(rows0, acc.at[dstb.at[CH - 1]], add=True)
    plsc.subcore_barrier()
    _dump_acc(acc, rows0, se_out, c, s)


_gpass = pl.kernel(
    _gpass_body,
    out_type=jax.ShapeDtypeStruct((NC, NPAD, H), _f32),
    mesh=_MESH,
    scratch_types=[
        pltpu.VMEM_SHARED((NPAD, H), _f32),
        pltpu.VMEM((K, H), _f32),
        pltpu.VMEM((K, H), _f32),
        pltpu.VMEM((EPT,), jnp.int32),
        pltpu.VMEM((CH, K), jnp.int32),
        pltpu.SemaphoreType.DMA,
        pltpu.SemaphoreType.DMA,
    ],
)


def _mlp1_body(h_ref, se_ref, sr_ref, cnt_ref, wa_ref, wb_ref, wc_ref,
               b_ref, o_ref):
    inv = 1.0 / jnp.maximum(cnt_ref[0, :, 0:1] + cnt_ref[1, :, 0:1], 1.0)
    acc = jnp.dot(h_ref[...], wa_ref[...], preferred_element_type=_f32)
    acc += jnp.dot(se_ref[0] * inv, wb_ref[0], preferred_element_type=_f32)
    acc += jnp.dot(se_ref[1] * inv, wb_ref[1], preferred_element_type=_f32)
    acc += jnp.dot(sr_ref[0] * inv, wc_ref[0], preferred_element_type=_f32)
    acc += jnp.dot(sr_ref[1] * inv, wc_ref[1], preferred_element_type=_f32)
    r = jnp.maximum(acc + b_ref[...], 0.0)
    o_ref[0] = r[:, :H]
    o_ref[1] = r[:, H:]


def _mlp2_body(h_ref, se_ref, sr_ref, cnt_ref, wa_ref, wb_ref, wc_ref,
               b_ref, o_ref):
    inv = 1.0 / jnp.maximum(cnt_ref[0, :, 0:1] + cnt_ref[1, :, 0:1], 1.0)
    acc = jnp.dot(h_ref[0], wa_ref[0], preferred_element_type=_f32)
    acc += jnp.dot(h_ref[1], wa_ref[1], preferred_element_type=_f32)
    acc += jnp.dot(se_ref[0] * inv, wb_ref[0], preferred_element_type=_f32)
    acc += jnp.dot(se_ref[1] * inv, wb_ref[1], preferred_element_type=_f32)
    acc += jnp.dot(sr_ref[0] * inv, wc_ref[0], preferred_element_type=_f32)
    acc += jnp.dot(sr_ref[1] * inv, wc_ref[1], preferred_element_type=_f32)
    o_ref[...] = jnp.maximum(acc + b_ref[...], 0.0)


_split_spec = pl.BlockSpec((NC, BN, H), lambda i: (0, i, 0))
_wsplit_spec = pl.BlockSpec((NC, H, EMB), lambda i: (0, 0, 0))
_b_spec = pl.BlockSpec((1, EMB), lambda i: (0, 0))

_mlp1 = pl.pallas_call(
    _mlp1_body,
    grid=(GRID,),
    in_specs=[
        pl.BlockSpec((BN, EMB), lambda i: (i, 0)),
        _split_spec,
        _split_spec,
        _split_spec,
        pl.BlockSpec((EMB, EMB), lambda i: (0, 0)),
        _wsplit_spec,
        _wsplit_spec,
        _b_spec,
    ],
    out_specs=_split_spec,
    out_shape=jax.ShapeDtypeStruct((NC, N, H), _f32),
)

_mlp2 = pl.pallas_call(
    _mlp2_body,
    grid=(GRID,),
    in_specs=[
        _split_spec,
        _split_spec,
        _split_spec,
        _split_spec,
        _wsplit_spec,
        _wsplit_spec,
        _wsplit_spec,
        _b_spec,
    ],
    out_specs=pl.BlockSpec((BN, EMB), lambda i: (i, 0)),
    out_shape=jax.ShapeDtypeStruct((N, EMB), _f32),
)


def kernel(h_e, h_r, edge_index, W1, b1, W2, b2):
    src = edge_index[0].astype(jnp.int32)
    dst = edge_index[1].astype(jnp.int32)
    src2 = jnp.concatenate([src, src + N])
    dst3 = dst.reshape(NS, CH, K)
    dstc = dst.reshape(NC * NS, CHC, KC)

    z128 = jnp.zeros((K, H), _f32)
    ones128 = jnp.ones((KC, H), _f32)

    tab1 = h_e.reshape(N, NC, H).transpose(1, 0, 2).reshape(NC * N, H)

    s_r = _rpass(h_r, dst3, z128)
    cnt = _cpass(dstc, ones128, z128)
    s_e1 = _gpass(tab1, src2, dst3, z128)

    w1a = W1[:EMB]
    w1b = W1[EMB:2 * EMB].reshape(NC, H, EMB)
    w1c = W1[2 * EMB:].reshape(NC, H, EMB)
    h1s = _mlp1(h_e, s_e1, s_r, cnt, w1a, w1b, w1c, b1.reshape(1, EMB))

    s_e2 = _gpass(h1s.reshape(NC * N, H), src2, dst3, z128)

    w2a = W2[:EMB].reshape(NC, H, EMB)
    w2b = W2[EMB:2 * EMB].reshape(NC, H, EMB)
    w2c = W2[2 * EMB:].reshape(NC, H, EMB)
    return _mlp2(h1s, s_e2, s_r, cnt, w2a, w2b, w2c, b2.reshape(1, EMB))


# speedup vs baseline: 1.3231x; 1.0436x over previous
import jax
import jax.numpy as jnp
from jax import lax
from jax.experimental import pallas as pl
from jax.experimental.pallas import tpu as pltpu
from jax.experimental.pallas import tpu_sc as plsc

N = 10000
E = 160000
EMB = 256
H = 128
NC = 2
NS = 16
NPAD = 10240
RPT = NPAD // NS
BN = 2000
GRID = N // BN

K = 80
EPT = E // NS
CH = EPT // K
KC = 125
EPW = E // (NC * NS)
CHC = EPW // KC

_MESH = plsc.VectorSubcoreMesh(core_axis_name="c", subcore_axis_name="s")
_f32 = jnp.float32


def _zero_acc(z_h, buf, acc, s):
    pltpu.sync_copy(z_h, buf)
    for t in range(RPT // K):
        pltpu.sync_copy(buf, acc.at[pl.ds(s * RPT + t * K, K)])


def _dump_acc(acc, buf, out, c, s):
    for t in range(RPT // K):
        pltpu.sync_copy(acc.at[pl.ds(s * RPT + t * K, K)], buf)
        pltpu.sync_copy(buf, out.at[c, pl.ds(s * RPT + t * K, K)])


def _rpass_body(hr_h, dst_h, z_h, sr_out, acc, rows0, rows1, dstb,
                sem0, sem1):
    c = lax.axis_index("c")
    s = lax.axis_index("s")
    _zero_acc(z_h, rows0, acc, s)
    pltpu.sync_copy(dst_h.at[s], dstb)
    plsc.subcore_barrier()

    def ld(j, buf, sem):
        src = hr_h.at[pl.ds(s * EPT + j * K, K), pl.ds(c * H, H)]
        pltpu.async_copy(src, buf, sem)

    def ldw(j, buf, sem):
        src = hr_h.at[pl.ds(s * EPT + j * K, K), pl.ds(c * H, H)]
        pltpu.make_async_copy(src, buf, sem).wait()

    ld(0, rows0, sem0)

    def step2(t, carry):
        j0 = 2 * t
        ld(j0 + 1, rows1, sem1)
        ldw(j0, rows0, sem0)
        pltpu.sync_copy(rows0, acc.at[dstb.at[j0]], add=True)
        ld(j0 + 2, rows0, sem0)
        ldw(j0 + 1, rows1, sem1)
        pltpu.sync_copy(rows1, acc.at[dstb.at[j0 + 1]], add=True)
        return carry

    lax.fori_loop(0, (CH - 1) // 2, step2, 0)
    ldw(CH - 1, rows0, sem0)
    pltpu.sync_copy(rows0, acc.at[dstb.at[CH - 1]], add=True)
    plsc.subcore_barrier()
    _dump_acc(acc, rows0, sr_out, c, s)


_rpass = pl.kernel(
    _rpass_body,
    out_type=jax.ShapeDtypeStruct((NC, NPAD, H), _f32),
    mesh=_MESH,
    scratch_types=[
        pltpu.VMEM_SHARED((NPAD, H), _f32),
        pltpu.VMEM((K, H), _f32),
        pltpu.VMEM((K, H), _f32),
        pltpu.VMEM((CH, K), jnp.int32),
        pltpu.SemaphoreType.DMA,
        pltpu.SemaphoreType.DMA,
    ],
)


def _cpass_body(dst_h, ones_h, z_h, cnt_out, acc, rows, ones_v, dstb, sem):
    c = lax.axis_index("c")
    s = lax.axis_index("s")
    w = c * NS + s
    _zero_acc(z_h, rows, acc, s)
    pltpu.sync_copy(ones_h, ones_v)
    pltpu.sync_copy(dst_h.at[w], dstb)
    plsc.subcore_barrier()

    def step(g, carry):
        for i in range(8):
            pltpu.async_copy(ones_v, acc.at[dstb.at[g * 8 + i]], sem, add=True)
        for i in range(8):
            pltpu.make_async_copy(ones_v, acc.at[dstb.at[g * 8 + i]], sem).wait()
        return carry

    lax.fori_loop(0, CHC // 8, step, 0)
    plsc.subcore_barrier()
    _dump_acc(acc, rows, cnt_out, c, s)


_cpass = pl.kernel(
    _cpass_body,
    out_type=jax.ShapeDtypeStruct((NC, NPAD, H), _f32),
    mesh=_MESH,
    scratch_types=[
        pltpu.VMEM_SHARED((NPAD, H), _f32),
        pltpu.VMEM((K, H), _f32),
        pltpu.VMEM((KC, H), _f32),
        pltpu.VMEM((CHC, KC), jnp.int32),
        pltpu.SemaphoreType.DMA,
    ],
)


def _gpass_body(tab_h, src_h, dst_h, z_h, se_out, acc, rows0, rows1,
                idxb, dstb, sem0, sem1):
    c = lax.axis_index("c")
    s = lax.axis_index("s")
    _zero_acc(z_h, rows0, acc, s)
    pltpu.sync_copy(src_h.at[pl.ds((c * NS + s) * EPT, EPT)], idxb)
    pltpu.sync_copy(dst_h.at[s], dstb)
    plsc.subcore_barrier()

    def g(j, buf, sem):
        pltpu.async_copy(tab_h.at[idxb.at[pl.ds(j * K, K)]], buf, sem)

    def gw(j, buf, sem):
        pltpu.make_async_copy(tab_h.at[idxb.at[pl.ds(j * K, K)]], buf, sem).wait()

    g(0, rows0, sem0)

    def step2(t, carry):
        j0 = 2 * t
        g(j0 + 1, rows1, sem1)
        gw(j0, rows0, sem0)
        pltpu.sync_copy(rows0, acc.at[dstb.at[j0]], add=True)
        g(j0 + 2, rows0, sem0)
        gw(j0 + 1, rows1, sem1)
        pltpu.sync_copy(rows1, acc.at[dstb.at[j0 + 1]], add=True)
        return carry

    lax.fori_loop(0, (CH - 1) // 2, step2, 0)
    gw(CH - 1, rows0, sem0)
    pltpu.sync_copy(rows0, acc.at[dstb.at[CH - 1]], add=True)
    plsc.subcore_barrier()
    _dump_acc(acc, rows0, se_out, c, s)


_gpass = pl.kernel(
    _gpass_body,
    out_type=jax.ShapeDtypeStruct((NC, NPAD, H), _f32),
    mesh=_MESH,
    scratch_types=[
        pltpu.VMEM_SHARED((NPAD, H), _f32),
        pltpu.VMEM((K, H), _f32),
        pltpu.VMEM((K, H), _f32),
        pltpu.VMEM((EPT,), jnp.int32),
        pltpu.VMEM((CH, K), jnp.int32),
        pltpu.SemaphoreType.DMA,
        pltpu.SemaphoreType.DMA,
    ],
)


def _mlp1_body(h_ref, se_ref, sr_ref, cnt_ref, wa_ref, wb_ref, wc_ref,
               b_ref, o_ref):
    inv = 1.0 / jnp.maximum(cnt_ref[0, :, 0:1] + cnt_ref[1, :, 0:1], 1.0)
    acc = jnp.dot(h_ref[...], wa_ref[...], preferred_element_type=_f32)
    acc += jnp.dot(se_ref[0] * inv, wb_ref[0], preferred_element_type=_f32)
    acc += jnp.dot(se_ref[1] * inv, wb_ref[1], preferred_element_type=_f32)
    acc += jnp.dot(sr_ref[0] * inv, wc_ref[0], preferred_element_type=_f32)
    acc += jnp.dot(sr_ref[1] * inv, wc_ref[1], preferred_element_type=_f32)
    r = jnp.maximum(acc + b_ref[...], 0.0)
    o_ref[0] = r[:, :H]
    o_ref[1] = r[:, H:]


def _mlp2_body(h_ref, se_ref, sr_ref, cnt_ref, wa_ref, wb_ref, wc_ref,
               b_ref, o_ref):
    inv = 1.0 / jnp.maximum(cnt_ref[0, :, 0:1] + cnt_ref[1, :, 0:1], 1.0)
    acc = jnp.dot(h_ref[0], wa_ref[0], preferred_element_type=_f32)
    acc += jnp.dot(h_ref[1], wa_ref[1], preferred_element_type=_f32)
    acc += jnp.dot(se_ref[0] * inv, wb_ref[0], preferred_element_type=_f32)
    acc += jnp.dot(se_ref[1] * inv, wb_ref[1], preferred_element_type=_f32)
    acc += jnp.dot(sr_ref[0] * inv, wc_ref[0], preferred_element_type=_f32)
    acc += jnp.dot(sr_ref[1] * inv, wc_ref[1], preferred_element_type=_f32)
    o_ref[...] = jnp.maximum(acc + b_ref[...], 0.0)


_split_spec = pl.BlockSpec((NC, BN, H), lambda i: (0, i, 0))
_cntn_spec = pl.BlockSpec((NC, BN, 8), lambda i: (0, i, 0))
_wsplit_spec = pl.BlockSpec((NC, H, EMB), lambda i: (0, 0, 0))
_b_spec = pl.BlockSpec((1, EMB), lambda i: (0, 0))

_mlp1 = pl.pallas_call(
    _mlp1_body,
    grid=(GRID,),
    in_specs=[
        pl.BlockSpec((BN, EMB), lambda i: (i, 0)),
        _split_spec,
        _split_spec,
        _cntn_spec,
        pl.BlockSpec((EMB, EMB), lambda i: (0, 0)),
        _wsplit_spec,
        _wsplit_spec,
        _b_spec,
    ],
    out_specs=_split_spec,
    out_shape=jax.ShapeDtypeStruct((NC, N, H), _f32),
)

_mlp2 = pl.pallas_call(
    _mlp2_body,
    grid=(GRID,),
    in_specs=[
        _split_spec,
        _split_spec,
        _split_spec,
        _cntn_spec,
        _wsplit_spec,
        _wsplit_spec,
        _wsplit_spec,
        _b_spec,
    ],
    out_specs=pl.BlockSpec((BN, EMB), lambda i: (i, 0)),
    out_shape=jax.ShapeDtypeStruct((N, EMB), _f32),
)


def kernel(h_e, h_r, edge_index, W1, b1, W2, b2):
    src = edge_index[0].astype(jnp.int32)
    dst = edge_index[1].astype(jnp.int32)
    src2 = jnp.concatenate([src, src + N])
    dst3 = dst.reshape(NS, CH, K)
    dstc = dst.reshape(NC * NS, CHC, KC)

    z128 = jnp.zeros((K, H), _f32)
    ones128 = jnp.ones((KC, H), _f32)

    tab1 = h_e.reshape(N, NC, H).transpose(1, 0, 2).reshape(NC * N, H)

    s_r = _rpass(h_r, dst3, z128)
    cnt = _cpass(dstc, ones128, z128)[:, :, :8]
    s_e1 = _gpass(tab1, src2, dst3, z128)

    w1a = W1[:EMB]
    w1b = W1[EMB:2 * EMB].reshape(NC, H, EMB)
    w1c = W1[2 * EMB:].reshape(NC, H, EMB)
    h1s = _mlp1(h_e, s_e1, s_r, cnt, w1a, w1b, w1c, b1.reshape(1, EMB))

    s_e2 = _gpass(h1s.reshape(NC * N, H), src2, dst3, z128)

    w2a = W2[:EMB].reshape(NC, H, EMB)
    w2b = W2[EMB:2 * EMB].reshape(NC, H, EMB)
    w2c = W2[2 * EMB:].reshape(NC, H, EMB)
    return _mlp2(h1s, s_e2, s_r, cnt, w2a, w2b, w2c, b2.reshape(1, EMB))


# speedup vs baseline: 1.4378x; 1.0867x over previous
import jax
import jax.numpy as jnp
from jax import lax
from jax.experimental import pallas as pl
from jax.experimental.pallas import tpu as pltpu
from jax.experimental.pallas import tpu_sc as plsc

N = 10000
E = 160000
EMB = 256
H = 128
NC = 2
NS = 16
NPAD = 10240
RPT = NPAD // NS
BN = 2000
GRID = N // BN

K = 80
EPT = E // NS
CH = EPT // K
KC = 125
EPW = E // (NC * NS)
CHC = EPW // KC

_MESH = plsc.VectorSubcoreMesh(core_axis_name="c", subcore_axis_name="s")
_f32 = jnp.float32


def _zero_acc(z_h, buf, acc, s):
    pltpu.sync_copy(z_h, buf)
    for t in range(RPT // K):
        pltpu.sync_copy(buf, acc.at[pl.ds(s * RPT + t * K, K)])


def _dump_acc(acc, buf, out, c, s):
    for t in range(RPT // K):
        pltpu.sync_copy(acc.at[pl.ds(s * RPT + t * K, K)], buf)
        pltpu.sync_copy(buf, out.at[c, pl.ds(s * RPT + t * K, K)])


def _rpass_body(hr_h, dst_h, z_h, sr_out, acc, rows0, rows1, dstb,
                sem0, sem1):
    c = lax.axis_index("c")
    s = lax.axis_index("s")
    _zero_acc(z_h, rows0, acc, s)
    pltpu.sync_copy(dst_h.at[s], dstb)
    plsc.subcore_barrier()

    def ld(j, buf, sem):
        src = hr_h.at[pl.ds(s * EPT + j * K, K), pl.ds(c * H, H)]
        pltpu.async_copy(src, buf, sem)

    def ldw(j, buf, sem):
        src = hr_h.at[pl.ds(s * EPT + j * K, K), pl.ds(c * H, H)]
        pltpu.make_async_copy(src, buf, sem).wait()

    ld(0, rows0, sem0)

    def step2(t, carry):
        j0 = 2 * t
        ld(j0 + 1, rows1, sem1)
        ldw(j0, rows0, sem0)
        pltpu.sync_copy(rows0, acc.at[dstb.at[j0]], add=True)
        ld(j0 + 2, rows0, sem0)
        ldw(j0 + 1, rows1, sem1)
        pltpu.sync_copy(rows1, acc.at[dstb.at[j0 + 1]], add=True)
        return carry

    lax.fori_loop(0, (CH - 1) // 2, step2, 0)
    ldw(CH - 1, rows0, sem0)
    pltpu.sync_copy(rows0, acc.at[dstb.at[CH - 1]], add=True)
    plsc.subcore_barrier()
    _dump_acc(acc, rows0, sr_out, c, s)


_rpass = pl.kernel(
    _rpass_body,
    out_type=jax.ShapeDtypeStruct((NC, NPAD, H), _f32),
    mesh=_MESH,
    scratch_types=[
        pltpu.VMEM_SHARED((NPAD, H), _f32),
        pltpu.VMEM((K, H), _f32),
        pltpu.VMEM((K, H), _f32),
        pltpu.VMEM((CH, K), jnp.int32),
        pltpu.SemaphoreType.DMA,
        pltpu.SemaphoreType.DMA,
    ],
)


def _cpass_body(dst_h, ones_h, z_h, cnt_out, acc, rows, ones_v, dstb, sem):
    c = lax.axis_index("c")
    s = lax.axis_index("s")
    w = c * NS + s
    _zero_acc(z_h, rows, acc, s)
    pltpu.sync_copy(ones_h, ones_v)
    pltpu.sync_copy(dst_h.at[w], dstb)
    plsc.subcore_barrier()

    def step(g, carry):
        for i in range(8):
            pltpu.async_copy(ones_v, acc.at[dstb.at[g * 8 + i]], sem, add=True)
        for i in range(8):
            pltpu.make_async_copy(ones_v, acc.at[dstb.at[g * 8 + i]], sem).wait()
        return carry

    lax.fori_loop(0, CHC // 8, step, 0)
    plsc.subcore_barrier()
    _dump_acc(acc, rows, cnt_out, c, s)


_cpass = pl.kernel(
    _cpass_body,
    out_type=jax.ShapeDtypeStruct((NC, NPAD, H), _f32),
    mesh=_MESH,
    scratch_types=[
        pltpu.VMEM_SHARED((NPAD, H), _f32),
        pltpu.VMEM((K, H), _f32),
        pltpu.VMEM((KC, H), _f32),
        pltpu.VMEM((CHC, KC), jnp.int32),
        pltpu.SemaphoreType.DMA,
    ],
)


HA = 64
HB = CH - HA


def _gpass_body(tab_h, src_h, dst_h, z_h, se_out, acc, rows0, rows1, rows2,
                idxb, dstb, sem0, sem1, sem2):
    c = lax.axis_index("c")
    s = lax.axis_index("s")
    base = (c * NS + s) * EPT
    _zero_acc(z_h, rows0, acc, s)
    plsc.subcore_barrier()

    bufs = (rows0, rows1, rows2)
    sems = (sem0, sem1, sem2)

    def g(j, b):
        pltpu.async_copy(tab_h.at[idxb.at[pl.ds(j * K, K)]], bufs[b], sems[b])

    def gw(j, b):
        pltpu.make_async_copy(tab_h.at[idxb.at[pl.ds(j * K, K)]],
                              bufs[b], sems[b]).wait()

    def sc(j, b):
        pltpu.sync_copy(bufs[b], acc.at[dstb.at[j]], add=True)

    def run_half(off, L):
        pltpu.sync_copy(src_h.at[pl.ds(base + off * K, L * K)],
                        idxb.at[pl.ds(0, L * K)])
        pltpu.sync_copy(dst_h.at[s, pl.ds(off, L)], dstb.at[pl.ds(0, L)])
        for b in range(3):
            g(b, b)
        T = (L - 6) // 3 + 1

        def step3(t, carry):
            j0 = 3 * t
            for b in range(3):
                gw(j0 + b, b)
                sc(j0 + b, b)
                g(j0 + b + 3, b)
            return carry

        lax.fori_loop(0, T, step3, 0)
        for j in range(3 * T, L):
            b = j % 3
            gw(j, b)
            sc(j, b)
            if j + 3 <= L - 1:
                g(j + 3, b)

    run_half(0, HA)
    run_half(HA, HB)
    plsc.subcore_barrier()
    _dump_acc(acc, rows0, se_out, c, s)


_gpass = pl.kernel(
    _gpass_body,
    out_type=jax.ShapeDtypeStruct((NC, NPAD, H), _f32),
    mesh=_MESH,
    scratch_types=[
        pltpu.VMEM_SHARED((NPAD, H), _f32),
        pltpu.VMEM((K, H), _f32),
        pltpu.VMEM((K, H), _f32),
        pltpu.VMEM((K, H), _f32),
        pltpu.VMEM((HA * K,), jnp.int32),
        pltpu.VMEM((HA, K), jnp.int32),
        pltpu.SemaphoreType.DMA,
        pltpu.SemaphoreType.DMA,
        pltpu.SemaphoreType.DMA,
    ],
)


def _mlp1_body(h_ref, se_ref, sr_ref, cnt_ref, wa_ref, wb_ref, wc_ref,
               b_ref, o_ref):
    inv = 1.0 / jnp.maximum(cnt_ref[0, :, 0:1] + cnt_ref[1, :, 0:1], 1.0)
    acc = jnp.dot(h_ref[...], wa_ref[...], preferred_element_type=_f32)
    acc += jnp.dot(se_ref[0] * inv, wb_ref[0], preferred_element_type=_f32)
    acc += jnp.dot(se_ref[1] * inv, wb_ref[1], preferred_element_type=_f32)
    acc += jnp.dot(sr_ref[0] * inv, wc_ref[0], preferred_element_type=_f32)
    acc += jnp.dot(sr_ref[1] * inv, wc_ref[1], preferred_element_type=_f32)
    r = jnp.maximum(acc + b_ref[...], 0.0)
    o_ref[0] = r[:, :H]
    o_ref[1] = r[:, H:]


def _mlp2_body(h_ref, se_ref, sr_ref, cnt_ref, wa_ref, wb_ref, wc_ref,
               b_ref, o_ref):
    inv = 1.0 / jnp.maximum(cnt_ref[0, :, 0:1] + cnt_ref[1, :, 0:1], 1.0)
    acc = jnp.dot(h_ref[0], wa_ref[0], preferred_element_type=_f32)
    acc += jnp.dot(h_ref[1], wa_ref[1], preferred_element_type=_f32)
    acc += jnp.dot(se_ref[0] * inv, wb_ref[0], preferred_element_type=_f32)
    acc += jnp.dot(se_ref[1] * inv, wb_ref[1], preferred_element_type=_f32)
    acc += jnp.dot(sr_ref[0] * inv, wc_ref[0], preferred_element_type=_f32)
    acc += jnp.dot(sr_ref[1] * inv, wc_ref[1], preferred_element_type=_f32)
    o_ref[...] = jnp.maximum(acc + b_ref[...], 0.0)


_split_spec = pl.BlockSpec((NC, BN, H), lambda i: (0, i, 0))
_cntn_spec = pl.BlockSpec((NC, BN, 8), lambda i: (0, i, 0))
_wsplit_spec = pl.BlockSpec((NC, H, EMB), lambda i: (0, 0, 0))
_b_spec = pl.BlockSpec((1, EMB), lambda i: (0, 0))

_mlp1 = pl.pallas_call(
    _mlp1_body,
    grid=(GRID,),
    in_specs=[
        pl.BlockSpec((BN, EMB), lambda i: (i, 0)),
        _split_spec,
        _split_spec,
        _cntn_spec,
        pl.BlockSpec((EMB, EMB), lambda i: (0, 0)),
        _wsplit_spec,
        _wsplit_spec,
        _b_spec,
    ],
    out_specs=_split_spec,
    out_shape=jax.ShapeDtypeStruct((NC, N, H), _f32),
)

_mlp2 = pl.pallas_call(
    _mlp2_body,
    grid=(GRID,),
    in_specs=[
        _split_spec,
        _split_spec,
        _split_spec,
        _cntn_spec,
        _wsplit_spec,
        _wsplit_spec,
        _wsplit_spec,
        _b_spec,
    ],
    out_specs=pl.BlockSpec((BN, EMB), lambda i: (i, 0)),
    out_shape=jax.ShapeDtypeStruct((N, EMB), _f32),
)


def kernel(h_e, h_r, edge_index, W1, b1, W2, b2):
    src = edge_index[0].astype(jnp.int32)
    dst = edge_index[1].astype(jnp.int32)
    src2 = jnp.concatenate([src, src + N])
    dst3 = dst.reshape(NS, CH, K)
    dstc = dst.reshape(NC * NS, CHC, KC)

    z128 = jnp.zeros((K, H), _f32)
    ones128 = jnp.ones((KC, H), _f32)

    tab1 = h_e.reshape(N, NC, H).transpose(1, 0, 2).reshape(NC * N, H)

    s_r = _rpass(h_r, dst3, z128)
    cnt = _cpass(dstc, ones128, z128)[:, :, :8]
    s_e1 = _gpass(tab1, src2, dst3, z128)

    w1a = W1[:EMB]
    w1b = W1[EMB:2 * EMB].reshape(NC, H, EMB)
    w1c = W1[2 * EMB:].reshape(NC, H, EMB)
    h1s = _mlp1(h_e, s_e1, s_r, cnt, w1a, w1b, w1c, b1.reshape(1, EMB))

    s_e2 = _gpass(h1s.reshape(NC * N, H), src2, dst3, z128)

    w2a = W2[:EMB].reshape(NC, H, EMB)
    w2b = W2[EMB:2 * EMB].reshape(NC, H, EMB)
    w2c = W2[2 * EMB:].reshape(NC, H, EMB)
    return _mlp2(h1s, s_e2, s_r, cnt, w2a, w2b, w2c, b2.reshape(1, EMB))


# speedup vs baseline: 1.5078x; 1.0486x over previous
import jax
import jax.numpy as jnp
from jax import lax
from jax.experimental import pallas as pl
from jax.experimental.pallas import tpu as pltpu
from jax.experimental.pallas import tpu_sc as plsc

N = 10000
E = 160000
EMB = 256
H = 128
NC = 2
NS = 16
NPAD = 10240
RPT = NPAD // NS
BN = 2000
GRID = N // BN

K = 80
EPT = E // NS
CH = EPT // K
KC = 125
EPW = E // (NC * NS)
CHC = EPW // KC

_MESH = plsc.VectorSubcoreMesh(core_axis_name="c", subcore_axis_name="s")
_f32 = jnp.float32


def _zero_acc(z_h, buf, acc, s):
    pltpu.sync_copy(z_h, buf)
    for t in range(RPT // K):
        pltpu.sync_copy(buf, acc.at[pl.ds(s * RPT + t * K, K)])


def _dump_acc(acc, buf, out, c, s):
    for t in range(RPT // K):
        pltpu.sync_copy(acc.at[pl.ds(s * RPT + t * K, K)], buf)
        pltpu.sync_copy(buf, out.at[c, pl.ds(s * RPT + t * K, K)])


def _rpass_body(hr_h, dst_h, z_h, sr_out, acc, rows0, rows1, rows2, dstb,
                sem0, sem1, sem2):
    c = lax.axis_index("c")
    s = lax.axis_index("s")
    _zero_acc(z_h, rows0, acc, s)
    pltpu.sync_copy(dst_h.at[s], dstb)
    plsc.subcore_barrier()

    bufs = (rows0, rows1, rows2)
    sems = (sem0, sem1, sem2)

    def ld(j, b):
        src = hr_h.at[pl.ds(s * EPT + j * K, K), pl.ds(c * H, H)]
        pltpu.async_copy(src, bufs[b], sems[b])

    def ldw(j, b):
        src = hr_h.at[pl.ds(s * EPT + j * K, K), pl.ds(c * H, H)]
        pltpu.make_async_copy(src, bufs[b], sems[b]).wait()

    for b in range(3):
        ld(b, b)
    T = (CH - 6) // 3 + 1

    def step3(t, carry):
        j0 = 3 * t
        for b in range(3):
            ldw(j0 + b, b)
            pltpu.sync_copy(bufs[b], acc.at[dstb.at[j0 + b]], add=True)
            ld(j0 + b + 3, b)
        return carry

    lax.fori_loop(0, T, step3, 0)
    for j in range(3 * T, CH):
        b = j % 3
        ldw(j, b)
        pltpu.sync_copy(bufs[b], acc.at[dstb.at[j]], add=True)
        if j + 3 <= CH - 1:
            ld(j + 3, b)
    plsc.subcore_barrier()
    _dump_acc(acc, rows0, sr_out, c, s)


_rpass = pl.kernel(
    _rpass_body,
    out_type=jax.ShapeDtypeStruct((NC, NPAD, H), _f32),
    mesh=_MESH,
    scratch_types=[
        pltpu.VMEM_SHARED((NPAD, H), _f32),
        pltpu.VMEM((K, H), _f32),
        pltpu.VMEM((K, H), _f32),
        pltpu.VMEM((K, H), _f32),
        pltpu.VMEM((CH, K), jnp.int32),
        pltpu.SemaphoreType.DMA,
        pltpu.SemaphoreType.DMA,
        pltpu.SemaphoreType.DMA,
    ],
)


def _cpass_body(dst_h, ones_h, z_h, cnt_out, acc, rows, ones_v, dstb, sem):
    c = lax.axis_index("c")
    s = lax.axis_index("s")
    w = c * NS + s
    _zero_acc(z_h, rows, acc, s)
    pltpu.sync_copy(ones_h, ones_v)
    pltpu.sync_copy(dst_h.at[w], dstb)
    plsc.subcore_barrier()

    def step(g, carry):
        for i in range(8):
            pltpu.async_copy(ones_v, acc.at[dstb.at[g * 8 + i]], sem, add=True)
        for i in range(8):
            pltpu.make_async_copy(ones_v, acc.at[dstb.at[g * 8 + i]], sem).wait()
        return carry

    lax.fori_loop(0, CHC // 8, step, 0)
    plsc.subcore_barrier()
    _dump_acc(acc, rows, cnt_out, c, s)


_cpass = pl.kernel(
    _cpass_body,
    out_type=jax.ShapeDtypeStruct((NC, NPAD, H), _f32),
    mesh=_MESH,
    scratch_types=[
        pltpu.VMEM_SHARED((NPAD, H), _f32),
        pltpu.VMEM((K, H), _f32),
        pltpu.VMEM((KC, H), _f32),
        pltpu.VMEM((CHC, KC), jnp.int32),
        pltpu.SemaphoreType.DMA,
    ],
)


HA = 64
HB = CH - HA


def _gpass_body(tab_h, src_h, dst_h, z_h, se_out, acc, rows0, rows1, rows2,
                idxb, dstb, sem0, sem1, sem2):
    c = lax.axis_index("c")
    s = lax.axis_index("s")
    base = (c * NS + s) * EPT
    _zero_acc(z_h, rows0, acc, s)
    plsc.subcore_barrier()

    bufs = (rows0, rows1, rows2)
    sems = (sem0, sem1, sem2)

    def g(j, b):
        pltpu.async_copy(tab_h.at[idxb.at[pl.ds(j * K, K)]], bufs[b], sems[b])

    def gw(j, b):
        pltpu.make_async_copy(tab_h.at[idxb.at[pl.ds(j * K, K)]],
                              bufs[b], sems[b]).wait()

    def sc(j, b):
        pltpu.sync_copy(bufs[b], acc.at[dstb.at[j]], add=True)

    def run_half(off, L):
        pltpu.sync_copy(src_h.at[pl.ds(base + off * K, L * K)],
                        idxb.at[pl.ds(0, L * K)])
        pltpu.sync_copy(dst_h.at[s, pl.ds(off, L)], dstb.at[pl.ds(0, L)])
        for b in range(3):
            g(b, b)
        T = (L - 6) // 3 + 1

        def step3(t, carry):
            j0 = 3 * t
            for b in range(3):
                gw(j0 + b, b)
                sc(j0 + b, b)
                g(j0 + b + 3, b)
            return carry

        lax.fori_loop(0, T, step3, 0)
        for j in range(3 * T, L):
            b = j % 3
            gw(j, b)
            sc(j, b)
            if j + 3 <= L - 1:
                g(j + 3, b)

    run_half(0, HA)
    run_half(HA, HB)
    plsc.subcore_barrier()
    _dump_acc(acc, rows0, se_out, c, s)


_gpass = pl.kernel(
    _gpass_body,
    out_type=jax.ShapeDtypeStruct((NC, NPAD, H), _f32),
    mesh=_MESH,
    scratch_types=[
        pltpu.VMEM_SHARED((NPAD, H), _f32),
        pltpu.VMEM((K, H), _f32),
        pltpu.VMEM((K, H), _f32),
        pltpu.VMEM((K, H), _f32),
        pltpu.VMEM((HA * K,), jnp.int32),
        pltpu.VMEM((HA, K), jnp.int32),
        pltpu.SemaphoreType.DMA,
        pltpu.SemaphoreType.DMA,
        pltpu.SemaphoreType.DMA,
    ],
)


def _mlp1_body(h_ref, se_ref, sr_ref, cnt_ref, wa_ref, wb_ref, wc_ref,
               b_ref, o_ref):
    inv = 1.0 / jnp.maximum(cnt_ref[0, :, 0:1] + cnt_ref[1, :, 0:1], 1.0)
    acc = jnp.dot(h_ref[...], wa_ref[...], preferred_element_type=_f32)
    acc += jnp.dot(se_ref[0] * inv, wb_ref[0], preferred_element_type=_f32)
    acc += jnp.dot(se_ref[1] * inv, wb_ref[1], preferred_element_type=_f32)
    acc += jnp.dot(sr_ref[0] * inv, wc_ref[0], preferred_element_type=_f32)
    acc += jnp.dot(sr_ref[1] * inv, wc_ref[1], preferred_element_type=_f32)
    r = jnp.maximum(acc + b_ref[...], 0.0)
    o_ref[0] = r[:, :H]
    o_ref[1] = r[:, H:]


def _mlp2_body(h_ref, se_ref, sr_ref, cnt_ref, wa_ref, wb_ref, wc_ref,
               b_ref, o_ref):
    inv = 1.0 / jnp.maximum(cnt_ref[0, :, 0:1] + cnt_ref[1, :, 0:1], 1.0)
    acc = jnp.dot(h_ref[0], wa_ref[0], preferred_element_type=_f32)
    acc += jnp.dot(h_ref[1], wa_ref[1], preferred_element_type=_f32)
    acc += jnp.dot(se_ref[0] * inv, wb_ref[0], preferred_element_type=_f32)
    acc += jnp.dot(se_ref[1] * inv, wb_ref[1], preferred_element_type=_f32)
    acc += jnp.dot(sr_ref[0] * inv, wc_ref[0], preferred_element_type=_f32)
    acc += jnp.dot(sr_ref[1] * inv, wc_ref[1], preferred_element_type=_f32)
    o_ref[...] = jnp.maximum(acc + b_ref[...], 0.0)


_split_spec = pl.BlockSpec((NC, BN, H), lambda i: (0, i, 0))
_cntn_spec = pl.BlockSpec((NC, BN, 8), lambda i: (0, i, 0))
_wsplit_spec = pl.BlockSpec((NC, H, EMB), lambda i: (0, 0, 0))
_b_spec = pl.BlockSpec((1, EMB), lambda i: (0, 0))

_mlp1 = pl.pallas_call(
    _mlp1_body,
    grid=(GRID,),
    in_specs=[
        pl.BlockSpec((BN, EMB), lambda i: (i, 0)),
        _split_spec,
        _split_spec,
        _cntn_spec,
        pl.BlockSpec((EMB, EMB), lambda i: (0, 0)),
        _wsplit_spec,
        _wsplit_spec,
        _b_spec,
    ],
    out_specs=_split_spec,
    out_shape=jax.ShapeDtypeStruct((NC, N, H), _f32),
)

_mlp2 = pl.pallas_call(
    _mlp2_body,
    grid=(GRID,),
    in_specs=[
        _split_spec,
        _split_spec,
        _split_spec,
        _cntn_spec,
        _wsplit_spec,
        _wsplit_spec,
        _wsplit_spec,
        _b_spec,
    ],
    out_specs=pl.BlockSpec((BN, EMB), lambda i: (i, 0)),
    out_shape=jax.ShapeDtypeStruct((N, EMB), _f32),
)


def kernel(h_e, h_r, edge_index, W1, b1, W2, b2):
    src = edge_index[0].astype(jnp.int32)
    dst = edge_index[1].astype(jnp.int32)
    src2 = jnp.concatenate([src, src + N])
    dst3 = dst.reshape(NS, CH, K)
    dstc = dst.reshape(NC * NS, CHC, KC)

    z128 = jnp.zeros((K, H), _f32)
    ones128 = jnp.ones((KC, H), _f32)

    tab1 = h_e.reshape(N, NC, H).transpose(1, 0, 2).reshape(NC * N, H)

    s_r = _rpass(h_r, dst3, z128)
    cnt = _cpass(dstc, ones128, z128)[:, :, :8]
    s_e1 = _gpass(tab1, src2, dst3, z128)

    w1a = W1[:EMB]
    w1b = W1[EMB:2 * EMB].reshape(NC, H, EMB)
    w1c = W1[2 * EMB:].reshape(NC, H, EMB)
    h1s = _mlp1(h_e, s_e1, s_r, cnt, w1a, w1b, w1c, b1.reshape(1, EMB))

    s_e2 = _gpass(h1s.reshape(NC * N, H), src2, dst3, z128)

    w2a = W2[:EMB].reshape(NC, H, EMB)
    w2b = W2[EMB:2 * EMB].reshape(NC, H, EMB)
    w2c = W2[2 * EMB:].reshape(NC, H, EMB)
    return _mlp2(h1s, s_e2, s_r, cnt, w2a, w2b, w2c, b2.reshape(1, EMB))
